# Initial kernel scaffold; baseline (speedup 1.0000x reference)
import jax
import jax.numpy as jnp
from jax.experimental import pallas as pl


def kernel(feats, w0, scale0, bias0, w1, scale1, bias1, w2, scale2, bias2, w3, scale3, bias3, wb, b_scale, b_bias):
    raise NotImplementedError("write your pallas kernel here")


# speedup vs baseline: 1.6089x; 1.6089x over previous
import functools
import math

import numpy as np
import jax
import jax.numpy as jnp
from jax.experimental import pallas as pl
from jax.experimental.pallas import tpu as pltpu

LEAKY_SLOPE = 0.01
VMEM_LIMIT_BYTES = 60 * 1024 * 1024


def _pool_matrix(H, W, s):
    P = np.zeros((s * s, H * W), np.float32)
    for i in range(s):
        r0, r1 = (i * H) // s, -((-(i + 1) * H) // s)
        for j in range(s):
            c0, c1 = (j * W) // s, -((-(j + 1) * W) // s)
            val = 1.0 / ((r1 - r0) * (c1 - c0))
            for rr in range(r0, r1):
                for cc in range(c0, c1):
                    P[i * s + j, rr * W + cc] = val
    return P


def _up_matrix(H, W, s):
    def axis_w(out_len, in_len):
        M = np.zeros((out_len, in_len), np.float32)
        for o in range(out_len):
            if in_len == 1:
                M[o, 0] = 1.0
                continue
            src = o * (in_len - 1) / (out_len - 1)
            i0 = min(int(math.floor(src)), in_len - 1)
            i1 = min(i0 + 1, in_len - 1)
            f = src - i0
            M[o, i0] += 1.0 - f
            M[o, i1] += f
        return M
    Wy, Wx = axis_w(H, s), axis_w(W, s)
    return np.einsum('yi,xj->yxij', Wy, Wx).reshape(H * W, s * s).astype(np.float32)


@functools.lru_cache(maxsize=None)
def _stage_constants(H, W, sizes, cout):
    s_tot = sum(s * s for s in sizes)
    P = np.zeros((s_tot, H * W), np.float32)
    B = np.zeros((H * W, s_tot), np.float32)
    M = np.zeros((s_tot, len(sizes) * cout), np.float32)
    off = 0
    for si, s in enumerate(sizes):
        P[off:off + s * s, :] = _pool_matrix(H, W, s)
        B[:, off:off + s * s] = _up_matrix(H, W, s)
        M[off:off + s * s, si * cout:(si + 1) * cout] = 1.0
        off += s * s
    return jnp.asarray(P), jnp.asarray(B), jnp.asarray(M)


def _psp_kernel(x_ref, p_ref, w1_ref, b1_ref, m_ref, up_ref, wu_ref, wx_ref,
                bb_ref, o_ref, acc_ref):
    _, H, W, Co = o_ref.shape
    x = x_ref[0]

    pooled = jnp.dot(p_ref[...], x, preferred_element_type=jnp.float32)
    y = jnp.dot(pooled, w1_ref[...], preferred_element_type=jnp.float32)
    y = y + b1_ref[...]
    y = jnp.where(y >= 0, y, LEAKY_SLOPE * y) * m_ref[...]
    up = jnp.dot(up_ref[...], y, preferred_element_type=jnp.float32)

    taps = (jnp.dot(up.astype(jnp.bfloat16), wu_ref[...],
                    preferred_element_type=jnp.float32) +
            jnp.dot(x.astype(jnp.bfloat16), wx_ref[...],
                    preferred_element_type=jnp.float32))

    acc_ref[...] = jnp.zeros_like(acc_ref)
    for dy in range(3):
        for dx in range(3):
            t = dy * 3 + dx
            tap = taps[:, t * Co:(t + 1) * Co].reshape(H, W, Co)
            acc_ref[pl.ds(2 - dy, H), pl.ds(2 - dx, W), :] += tap
    out = acc_ref[pl.ds(1, H), pl.ds(1, W), :] + bb_ref[0]
    o_ref[0] = jnp.where(out >= 0, out, LEAKY_SLOPE * out)


def kernel(feats, w0, scale0, bias0, w1, scale1, bias1, w2, scale2, bias2,
           w3, scale3, bias3, wb, b_scale, b_bias):
    sizes = (1, 2, 3, 6)
    N, Cin, H, W = feats.shape
    HW = H * W
    Cout = w0.shape[1]
    nCo = len(sizes) * Cout
    S = sum(s * s for s in sizes)

    x_flat = jnp.transpose(feats, (0, 2, 3, 1)).reshape(N, HW, Cin)

    p_all, b_all, mask = _stage_constants(H, W, sizes, Cout)
    w_cat = jnp.concatenate([w0 * scale0, w1 * scale1, w2 * scale2, w3 * scale3],
                            axis=1)
    bias_cat = jnp.concatenate([bias0, bias1, bias2, bias3], axis=1)

    Ct = nCo + Cin
    w_big = jnp.transpose(wb.reshape(9, Ct, Cout), (1, 0, 2)).reshape(Ct, 9 * Cout)
    w_big = (w_big * jnp.tile(b_scale, (1, 9))).astype(jnp.bfloat16)
    wu, wx = w_big[:nCo], w_big[nCo:]

    flops = 2 * N * (S * HW * Cin + S * Cin * nCo + HW * S * nCo
                     + 9 * HW * Ct * Cout)
    bytes_accessed = 4 * (N * HW * Cin + S * HW + Cin * nCo + S * nCo + HW * S
                          + N * HW * Cout) + 2 * Ct * 9 * Cout

    out_nhwc = pl.pallas_call(
        _psp_kernel,
        out_shape=jax.ShapeDtypeStruct((N, H, W, Cout), jnp.float32),
        grid=(N,),
        in_specs=[
            pl.BlockSpec((1, HW, Cin), lambda n: (n, 0, 0)),
            pl.BlockSpec((S, HW), lambda n: (0, 0)),
            pl.BlockSpec((Cin, nCo), lambda n: (0, 0)),
            pl.BlockSpec((1, nCo), lambda n: (0, 0)),
            pl.BlockSpec((S, nCo), lambda n: (0, 0)),
            pl.BlockSpec((HW, S), lambda n: (0, 0)),
            pl.BlockSpec((nCo, 9 * Cout), lambda n: (0, 0)),
            pl.BlockSpec((Cin, 9 * Cout), lambda n: (0, 0)),
            pl.BlockSpec((1, Cout), lambda n: (0, 0)),
        ],
        out_specs=pl.BlockSpec((1, H, W, Cout), lambda n: (n, 0, 0, 0)),
        scratch_shapes=[pltpu.VMEM((H + 2, W + 2, Cout), jnp.float32)],
        compiler_params=pltpu.CompilerParams(
            dimension_semantics=("parallel",),
            vmem_limit_bytes=VMEM_LIMIT_BYTES),
        cost_estimate=pl.CostEstimate(flops=flops, transcendentals=0,
                                      bytes_accessed=bytes_accessed),
    )(x_flat, p_all, w_cat, bias_cat, mask, b_all, wu, wx, b_bias)

    return jnp.transpose(out_nhwc, (0, 3, 1, 2))
